# baseline (speedup 1.0000x reference)
import jax
import jax.numpy as jnp
from jax import lax
from jax.experimental import pallas as pl
from jax.experimental.pallas import tpu as pltpu
from jax.experimental.pallas import tpu_sc as plsc

F32 = jnp.float32

NC = 2
NS = 16
NW = NC * NS
CH = 128
SUB = 32


def _sc_pass(compute, cwidth, nchunks, n_pad):
    rpt = n_pad // NS
    zr = 40
    assert rpt % zr == 0

    mesh = plsc.VectorSubcoreMesh(core_axis_name="c", subcore_axis_name="s")
    out_type = jax.ShapeDtypeStruct((NC, n_pad, 128), F32)
    NSUB = CH // SUB
    scratch = (
        pltpu.MemorySpace.VMEM((2, 1, CH), jnp.int32),
        pltpu.MemorySpace.VMEM((2, 1, CH), jnp.int32),
        pltpu.MemorySpace.VMEM((2, SUB, 128), F32),
        pltpu.MemorySpace.VMEM((2, SUB, 128), F32),
        pltpu.MemorySpace.VMEM((2, SUB, cwidth), F32),
        pltpu.MemorySpace.VMEM((CH, 128), F32),
        pltpu.MemorySpace.VMEM((zr, 128), F32),
        pltpu.MemorySpace.VMEM_SHARED((n_pad, 128), F32),
        pltpu.SemaphoreType.DMA,
        pltpu.SemaphoreType.DMA,
        pltpu.SemaphoreType.DMA,
    )

    def body(a_h, b_h, c_h, src_h, dst_h, u_out,
             si, di, a_v, b_v, c_v, t_v, zb, u_sh, s0, s1, s2):
        c = lax.axis_index("c")
        s = lax.axis_index("s")
        wid = s * NC + c
        sems = (s0, s1)

        def issue(kk, i4, par):
            off = (wid * nchunks + kk) * CH + i4 * SUB
            kp = kk % 2
            if i4 == 0:
                pltpu.sync_copy(src_h.at[pl.ds(off, CH)], si.at[kp, 0])
                pltpu.sync_copy(dst_h.at[pl.ds(off, CH)], di.at[kp, 0])
            pltpu.async_copy(a_h.at[di.at[kp, 0, pl.ds(i4 * SUB, SUB)]],
                             a_v.at[par], sems[par])
            pltpu.async_copy(b_h.at[si.at[kp, 0, pl.ds(i4 * SUB, SUB)]],
                             b_v.at[par], sems[par])
            pltpu.async_copy(c_h.at[pl.ds(off, SUB)], c_v.at[par], sems[par])

        def wait(par):
            pltpu.make_async_copy(a_h.at[pl.ds(0, SUB)], a_v.at[par],
                                  sems[par]).wait()
            pltpu.make_async_copy(b_h.at[pl.ds(0, SUB)], b_v.at[par],
                                  sems[par]).wait()
            pltpu.make_async_copy(c_h.at[pl.ds(0, SUB)], c_v.at[par],
                                  sems[par]).wait()

        @pl.loop(0, zr)
        def _(r):
            for j in range(8):
                zb[r, pl.ds(j * 16, 16)] = jnp.zeros((16,), F32)

        for j in range(rpt // zr):
            pltpu.sync_copy(zb, u_sh.at[pl.ds(s * rpt + j * zr, zr)])

        plsc.subcore_barrier()

        issue(0, 0, 0)

        @pl.loop(0, nchunks)
        def _(k):
            for i4 in range(NSUB):
                par = i4 % 2
                if i4 + 1 < NSUB:
                    issue(k, i4 + 1, 1 - par)
                else:
                    @pl.when(k + 1 < nchunks)
                    def _():
                        issue(k + 1, 0, 1 - par)
                wait(par)
                if i4 == 0:
                    @pl.when(k > 0)
                    def _():
                        pltpu.make_async_copy(
                            t_v, u_sh.at[pl.ds(0, CH)], s2).wait()

                @pl.loop(0, SUB, unroll=4)
                def _(r, _i4=i4, _par=par):
                    for j in range(8):
                        t_v[_i4 * SUB + r, pl.ds(j * 16, 16)] = compute(
                            r, j, a_v.at[_par], b_v.at[_par], c_v.at[_par])

            pltpu.async_copy(t_v, u_sh.at[di.at[k % 2, 0]], s2, add=True)

        pltpu.make_async_copy(t_v, u_sh.at[pl.ds(0, CH)], s2).wait()
        plsc.subcore_barrier()
        row = pl.ds(s * rpt, rpt)
        pltpu.sync_copy(u_sh.at[row], u_out.at[c, row])

    return pl.kernel(body, out_type=out_type, mesh=mesh,
                     scratch_types=scratch)


def _conv1_compute(r, j, a_v, b_v, c_v):
    sl = pl.ds(j * 16, 16)
    if j < 2:
        return jnp.maximum(
            a_v[r, sl] + b_v[r, pl.ds(32 + j * 16, 16)] + c_v[r, sl], 0.0)
    if j == 6:
        lane = lax.iota(jnp.int32, 16).astype(F32)
        lane1 = jnp.maximum(1.0 - jnp.abs(lane - 1.0), 0.0)
        return c_v[r, sl] + lane1
    return c_v[r, sl]


def _conv2_compute(r, j, a_v, b_v, c_v):
    sl = pl.ds(j * 16, 16)
    if j < 4:
        return jnp.maximum(
            a_v[r, sl] + b_v[r, pl.ds(64 + j * 16, 16)] + c_v[r, sl], 0.0)
    return jnp.zeros((16,), F32)


def _conv3_compute(r, j, a_v, b_v, c_v):
    sl = pl.ds(j * 16, 16)
    return jnp.maximum(a_v[r, sl] + b_v[r, sl] + c_v[r, sl], 0.0)


def _full(shape):
    nd = len(shape)
    return pl.BlockSpec(shape, lambda i, _nd=nd: (0,) * _nd)


def _ln_tc(x, g, b, eps=1e-5):
    m = jnp.mean(x, axis=-1, keepdims=True)
    v = jnp.mean((x - m) ** 2, axis=-1, keepdims=True)
    return (x - m) * lax.rsqrt(v + eps) * g + b


def _dot(a, b):
    return jnp.dot(a, b, preferred_element_type=F32)


def _enc_front(ea, lng, lnb, w1, b1, cw1, cb1, cw2, cb2):
    ln = _ln_tc(ea, lng[...], lnb[...])
    h = jnp.maximum(_dot(ln, w1[...]) + b1[...], 0.0)
    t = jnp.maximum(_dot(ea, cw1[...]) + cb1[...], 0.0)
    ew = jax.nn.sigmoid(_dot(t, cw2[...]) + cb2[...])
    return h * ew, ew


_ENC_FRONT_SPECS = [(1, 16), (1, 16), (16, 64), (1, 64),
                    (16, 8), (1, 8), (8, 1), (1, 1)]


def _enc1_call(e_pad, tile):
    def body(ea_ref, lng, lnb, w1, b1, cw1, cb1, cw2, cb2,
             wc1, bc1, b11, cc_ref):
        ea = ea_ref[...]
        hw, ew = _enc_front(ea, lng, lnb, w1, b1, cw1, cb1, cw2, cb2)
        c1 = _dot(hw, wc1[...]) + bc1[...] * ew + b11[...]
        cc_ref[...] = jnp.concatenate(
            [c1, hw, ew, jnp.zeros((ea.shape[0], 31), F32)], axis=1)

    grid = (e_pad // tile,)
    in_specs = [pl.BlockSpec((tile, 16), lambda i: (i, 0))] + [
        _full(s) for s in _ENC_FRONT_SPECS + [(64, 32), (1, 32), (1, 32)]]
    out_specs = [pl.BlockSpec((tile, 128), lambda i: (i, 0))]
    out_shape = [jax.ShapeDtypeStruct((e_pad, 128), F32)]
    return pl.pallas_call(body, grid=grid, in_specs=in_specs,
                          out_specs=out_specs, out_shape=out_shape)


def _enc23_call(e_pad, tile):
    def body(ea_ref, lng, lnb, w1, b1, cw1, cb1, cw2, cb2,
             wc2e, bc2e, b12, wc3, bc3, b13, c2_ref, c3_ref):
        ea = ea_ref[...]
        hw, ew = _enc_front(ea, lng, lnb, w1, b1, cw1, cb1, cw2, cb2)
        c2_ref[...] = _dot(hw, wc2e[...]) + bc2e[...] * ew + b12[...]
        c3_ref[...] = _dot(hw, wc3[...]) + bc3[...] * ew + b13[...]

    grid = (e_pad // tile,)
    in_specs = [pl.BlockSpec((tile, 16), lambda i: (i, 0))] + [
        _full(s) for s in _ENC_FRONT_SPECS + [(64, 64), (1, 64), (1, 64),
                                              (64, 128), (1, 128), (1, 128)]]
    out_specs = [pl.BlockSpec((tile, w), lambda i: (i, 0))
                 for w in (64, 128)]
    out_shape = [jax.ShapeDtypeStruct((e_pad, w), F32) for w in (64, 128)]
    return pl.pallas_call(body, grid=grid, in_specs=in_specs,
                          out_specs=out_specs, out_shape=out_shape)


def _node_pre_call(n_pad, blk):
    def body(x_ref, bg, bb, bm, bv, wab, skw, skb, ab_ref, xp_ref):
        x = (x_ref[...] - bm[...]) * lax.rsqrt(bv[...] + 1e-5) * bg[...] \
            + bb[...]
        ab_ref[...] = _dot(x, wab[...])
        xp_ref[...] = 0.01 * (_dot(x, skw[...]) + skb[...])

    grid = (n_pad // blk,)
    in_specs = [pl.BlockSpec((blk, 128), lambda i: (i, 0))] + [
        _full(s) for s in [(1, 128)] * 4 + [(128, 128), (128, 128),
                                            (1, 128)]]
    out_specs = [pl.BlockSpec((blk, 128), lambda i: (i, 0))] * 2
    out_shape = [jax.ShapeDtypeStruct((n_pad, 128), F32)] * 2
    return pl.pallas_call(body, grid=grid, in_specs=in_specs,
                          out_specs=out_specs, out_shape=out_shape)


def _mid1_call(n_pad, blk):
    def body(u_ref, w2, b2, lng, lnb, wab, ew2, eb2, ab_ref, cnt_ref,
             efs_ref):
        ut = u_ref[0] + u_ref[1]
        u = ut[:, 0:32]
        uhw = ut[:, 32:96]
        sew = ut[:, 96:97]
        cnt = ut[:, 97:98]
        efs_ref[...] = _dot(uhw, ew2[...]) + eb2[...] * sew
        deg = jnp.maximum(cnt, 1.0)
        conv = _dot(u / deg, w2[...]) + b2[...] * jnp.minimum(cnt, 1.0)
        xn = _ln_tc(conv, lng[...], lnb[...])
        xn = jnp.maximum(xn, 0.01 * xn)
        ab_ref[...] = _dot(xn, wab[...])
        cnt_ref[...] = cnt

    grid = (n_pad // blk,)
    in_specs = [pl.BlockSpec((NC, blk, 128), lambda i: (0, i, 0))] + [
        _full(s) for s in [(32, 32), (1, 32), (1, 32), (1, 32), (32, 128),
                           (64, 128), (1, 128)]]
    out_specs = [pl.BlockSpec((blk, 128), lambda i: (i, 0)),
                 pl.BlockSpec((blk, 1), lambda i: (i, 0)),
                 pl.BlockSpec((blk, 128), lambda i: (i, 0))]
    out_shape = [jax.ShapeDtypeStruct((n_pad, 128), F32),
                 jax.ShapeDtypeStruct((n_pad, 1), F32),
                 jax.ShapeDtypeStruct((n_pad, 128), F32)]
    return pl.pallas_call(body, grid=grid, in_specs=in_specs,
                          out_specs=out_specs, out_shape=out_shape)


def _mid2_call(n_pad, blk):
    def body(u_ref, cnt_ref, w2, b2, lng, lnb, wd, ws, a_ref, b_ref):
        u = (u_ref[0] + u_ref[1])[:, 0:64]
        cnt = cnt_ref[...]
        deg = jnp.maximum(cnt, 1.0)
        conv = _dot(u / deg, w2[...]) + b2[...] * jnp.minimum(cnt, 1.0)
        xn = jnp.maximum(_ln_tc(conv, lng[...], lnb[...]), 0.0)
        a_ref[...] = _dot(xn, wd[...])
        b_ref[...] = _dot(xn, ws[...])

    grid = (n_pad // blk,)
    in_specs = [pl.BlockSpec((NC, blk, 128), lambda i: (0, i, 0)),
                pl.BlockSpec((blk, 1), lambda i: (i, 0))] + [
        _full(s) for s in [(64, 64), (1, 64), (1, 64), (1, 64),
                           (64, 128), (64, 128)]]
    out_specs = [pl.BlockSpec((blk, 128), lambda i: (i, 0))] * 2
    out_shape = [jax.ShapeDtypeStruct((n_pad, 128), F32)] * 2
    return pl.pallas_call(body, grid=grid, in_specs=in_specs,
                          out_specs=out_specs, out_shape=out_shape)


def _final_call(n_pad, blk):
    def body(u_ref, cnt_ref, efs_ref, xp_ref, w2, b2, lng, lnb,
             nw1, nb1, nw2, nb2, nw3, nb3, xf_ref, pr_ref):
        u = u_ref[0] + u_ref[1]
        cnt = cnt_ref[...]
        deg = jnp.maximum(cnt, 1.0)
        conv = _dot(u / deg, w2[...]) + b2[...] * jnp.minimum(cnt, 1.0)
        x3 = jnp.maximum(_ln_tc(conv, lng[...], lnb[...]), 0.0)
        xf = jnp.concatenate([xp_ref[...] + x3, efs_ref[...] / deg], axis=1)
        xf_ref[...] = xf
        h = _dot(xf, nw1[...]) + nb1[...]
        h = jnp.where(h > 0, h, jnp.exp(jnp.minimum(h, 0.0)) - 1.0)
        h = _dot(h, nw2[...]) + nb2[...]
        h = jnp.where(h > 0, h, jnp.exp(jnp.minimum(h, 0.0)) - 1.0)
        pr_ref[...] = _dot(h, nw3[...]) + nb3[...]

    grid = (n_pad // blk,)
    in_specs = [pl.BlockSpec((NC, blk, 128), lambda i: (0, i, 0)),
                pl.BlockSpec((blk, 1), lambda i: (i, 0)),
                pl.BlockSpec((blk, 128), lambda i: (i, 0)),
                pl.BlockSpec((blk, 128), lambda i: (i, 0))] + [
        _full(s) for s in [(128, 128), (1, 128), (1, 128), (1, 128),
                           (256, 128), (1, 128), (128, 64), (1, 64),
                           (64, 1), (1, 1)]]
    out_specs = [pl.BlockSpec((blk, 256), lambda i: (i, 0)),
                 pl.BlockSpec((blk, 1), lambda i: (i, 0))]
    out_shape = [jax.ShapeDtypeStruct((n_pad, 256), F32),
                 jax.ShapeDtypeStruct((n_pad, 1), F32)]
    return pl.pallas_call(body, grid=grid, in_specs=in_specs,
                          out_specs=out_specs, out_shape=out_shape)


@jax.jit
def _run(x_in, edge_index, edge_attr, params):
    p = params
    n, _ = x_in.shape
    e = edge_index.shape[1]

    epc = NW * CH * 8
    e_pad = ((e + epc - 1) // epc) * epc
    n_pad = ((n + 8 + 639) // 640) * 640
    nchunks = e_pad // (NW * CH)

    src = jnp.concatenate(
        [edge_index[0], jnp.full((e_pad - e,), n, jnp.int32)])
    dst = jnp.concatenate(
        [edge_index[1], jnp.full((e_pad - e,), n, jnp.int32)])
    ea = jnp.pad(edge_attr, ((0, e_pad - e), (0, 0)))
    x_pad = jnp.pad(x_in, ((0, n_pad - n), (0, 0)))

    def r2(v):
        return v.reshape(1, -1)

    def padw(w, cols):
        return jnp.pad(w, ((0, 0), (0, cols - w.shape[1])))

    wab1 = padw(jnp.concatenate([p['c1_W1'][0:128], p['c1_W1'][128:256]], 1),
                128)
    wab2 = jnp.concatenate([p['c2_W1'][0:32], p['c2_W1'][32:64]], 1)

    enc_w2 = p['enc_W2']
    enc_b2 = r2(p['enc_b2'])

    def hdot(a, b):
        return jnp.dot(a, b, precision=lax.Precision.HIGHEST)

    we1 = hdot(enc_w2, p['c1_W1'][256:384])
    we2 = hdot(enc_w2, p['c2_W1'][64:192])
    we3 = hdot(enc_w2, p['c3_W1'][128:256])
    be1 = hdot(enc_b2, p['c1_W1'][256:384])
    be2 = hdot(enc_b2, p['c2_W1'][64:192])
    be3 = hdot(enc_b2, p['c3_W1'][128:256])

    enc_front_args = (
        ea, r2(p['enc_ln_g']), r2(p['enc_ln_b']),
        p['enc_W1'], r2(p['enc_b1']),
        p['cls_W1'], r2(p['cls_b1']), p['cls_W2'], r2(p['cls_b2']))
    cc1 = _enc1_call(e_pad, 1024)(
        *enc_front_args, we1, be1, r2(p['c1_b1']))[0]
    c2, c3 = _enc23_call(e_pad, 1024)(
        *enc_front_args, we2, be2, r2(p['c2_b1']), we3, be3, r2(p['c3_b1']))

    blk = n_pad // 8
    ab1, xp = _node_pre_call(n_pad, blk)(
        x_pad, r2(p['bn0_g']), r2(p['bn0_b']), r2(p['bn0_m']),
        r2(p['bn0_v']), wab1, p['skip_W'], r2(p['skip_b']))

    u1 = _sc_pass(_conv1_compute, 128, nchunks, n_pad)(
        ab1, ab1, cc1, src, dst)

    ab2, cnt, efs = _mid1_call(n_pad, blk)(
        u1, p['c1_W2'], r2(p['c1_b2']), r2(p['ln1_g']), r2(p['ln1_b']),
        wab2, enc_w2, enc_b2)

    u2 = _sc_pass(_conv2_compute, 64, nchunks, n_pad)(
        ab2, ab2, c2, src, dst)

    a3, b3 = _mid2_call(n_pad, blk)(
        u2, cnt, p['c2_W2'], r2(p['c2_b2']), r2(p['ln2_g']), r2(p['ln2_b']),
        p['c3_W1'][0:64], p['c3_W1'][64:128])

    u3 = _sc_pass(_conv3_compute, 128, nchunks, n_pad)(
        a3, b3, c3, src, dst)

    xf, probs = _final_call(n_pad, blk)(
        u3, cnt, efs, xp, p['c3_W2'], r2(p['c3_b2']),
        r2(p['ln3_g']), r2(p['ln3_b']),
        p['np_W1'], r2(p['np_b1']), p['np_W2'], r2(p['np_b2']),
        p['np_W3'], r2(p['np_b3']))

    return xf[:n], probs[:n]


def kernel(x_in, edge_index, edge_attr, params):
    xf, probs = _run(x_in, edge_index, edge_attr, params)
    return (xf, probs, None, None, None)


# speedup vs baseline: 1.0491x; 1.0491x over previous
import jax
import jax.numpy as jnp
from jax import lax
from jax.experimental import pallas as pl
from jax.experimental.pallas import tpu as pltpu
from jax.experimental.pallas import tpu_sc as plsc

F32 = jnp.float32

NC = 2
NS = 16
NW = NC * NS
CH = 128
SUB = 32


def _sc_pass(compute, cwidth, nchunks, n_pad):
    rpt = n_pad // NS
    zr = 40
    assert rpt % zr == 0

    mesh = plsc.VectorSubcoreMesh(core_axis_name="c", subcore_axis_name="s")
    out_type = jax.ShapeDtypeStruct((NC, n_pad, 128), F32)
    NSUB = CH // SUB
    scratch = (
        pltpu.MemorySpace.VMEM((2, 1, CH), jnp.int32),
        pltpu.MemorySpace.VMEM((2, 1, CH), jnp.int32),
        pltpu.MemorySpace.VMEM((2, SUB, 128), F32),
        pltpu.MemorySpace.VMEM((2, SUB, 128), F32),
        pltpu.MemorySpace.VMEM((2, SUB, cwidth), F32),
        pltpu.MemorySpace.VMEM((CH, 128), F32),
        pltpu.MemorySpace.VMEM((zr, 128), F32),
        pltpu.MemorySpace.VMEM_SHARED((n_pad, 128), F32),
        pltpu.SemaphoreType.DMA,
        pltpu.SemaphoreType.DMA,
        pltpu.SemaphoreType.DMA,
    )

    def body(a_h, b_h, c_h, src_h, dst_h, u_out,
             si, di, a_v, b_v, c_v, t_v, zb, u_sh, s0, s1, s2):
        c = lax.axis_index("c")
        s = lax.axis_index("s")
        wid = s * NC + c
        sems = (s0, s1)

        def issue(kk, i4, par):
            off = (wid * nchunks + kk) * CH + i4 * SUB
            kp = kk % 2
            if i4 == 0:
                pltpu.sync_copy(src_h.at[pl.ds(off, CH)], si.at[kp, 0])
                pltpu.sync_copy(dst_h.at[pl.ds(off, CH)], di.at[kp, 0])
            pltpu.async_copy(a_h.at[di.at[kp, 0, pl.ds(i4 * SUB, SUB)]],
                             a_v.at[par], sems[par])
            pltpu.async_copy(b_h.at[si.at[kp, 0, pl.ds(i4 * SUB, SUB)]],
                             b_v.at[par], sems[par])
            pltpu.async_copy(c_h.at[pl.ds(off, SUB)], c_v.at[par], sems[par])

        def wait(par):
            pltpu.make_async_copy(a_h.at[pl.ds(0, SUB)], a_v.at[par],
                                  sems[par]).wait()
            pltpu.make_async_copy(b_h.at[pl.ds(0, SUB)], b_v.at[par],
                                  sems[par]).wait()
            pltpu.make_async_copy(c_h.at[pl.ds(0, SUB)], c_v.at[par],
                                  sems[par]).wait()

        @pl.loop(0, zr)
        def _(r):
            for j in range(8):
                zb[r, pl.ds(j * 16, 16)] = jnp.zeros((16,), F32)

        for j in range(rpt // zr):
            pltpu.sync_copy(zb, u_sh.at[pl.ds(s * rpt + j * zr, zr)])

        plsc.subcore_barrier()

        issue(0, 0, 0)

        @pl.loop(0, nchunks)
        def _(k):
            for i4 in range(NSUB):
                par = i4 % 2
                if i4 + 1 < NSUB:
                    issue(k, i4 + 1, 1 - par)
                else:
                    @pl.when(k + 1 < nchunks)
                    def _():
                        issue(k + 1, 0, 1 - par)
                wait(par)
                if i4 == 0:
                    @pl.when(k > 0)
                    def _():
                        pltpu.make_async_copy(
                            t_v, u_sh.at[pl.ds(0, CH)], s2).wait()

                @pl.loop(0, SUB, unroll=4)
                def _(r, _i4=i4, _par=par):
                    for j in range(8):
                        t_v[_i4 * SUB + r, pl.ds(j * 16, 16)] = compute(
                            r, j, a_v.at[_par], b_v.at[_par], c_v.at[_par])

            pltpu.async_copy(t_v, u_sh.at[di.at[k % 2, 0]], s2, add=True)

        pltpu.make_async_copy(t_v, u_sh.at[pl.ds(0, CH)], s2).wait()
        plsc.subcore_barrier()
        row = pl.ds(s * rpt, rpt)
        pltpu.sync_copy(u_sh.at[row], u_out.at[c, row])

    return pl.kernel(body, out_type=out_type, mesh=mesh,
                     scratch_types=scratch)


def _conv1_compute(r, j, a_v, b_v, c_v):
    sl = pl.ds(j * 16, 16)
    if j < 2:
        return jnp.maximum(
            a_v[r, sl] + b_v[r, pl.ds(32 + j * 16, 16)] + c_v[r, sl], 0.0)
    if j == 6:
        lane = lax.iota(jnp.int32, 16).astype(F32)
        lane1 = jnp.maximum(1.0 - jnp.abs(lane - 1.0), 0.0)
        return c_v[r, sl] + lane1
    return c_v[r, sl]


def _conv2_compute(r, j, a_v, b_v, c_v):
    sl = pl.ds(j * 16, 16)
    if j < 4:
        return jnp.maximum(
            a_v[r, sl] + b_v[r, pl.ds(64 + j * 16, 16)] + c_v[r, sl], 0.0)
    return jnp.zeros((16,), F32)


def _conv3_compute(r, j, a_v, b_v, c_v):
    sl = pl.ds(j * 16, 16)
    return jnp.maximum(a_v[r, sl] + b_v[r, sl] + c_v[r, sl], 0.0)


def _full(shape):
    nd = len(shape)
    return pl.BlockSpec(shape, lambda i, _nd=nd: (0,) * _nd)


def _ln_tc(x, g, b, eps=1e-5):
    m = jnp.mean(x, axis=-1, keepdims=True)
    v = jnp.mean((x - m) ** 2, axis=-1, keepdims=True)
    return (x - m) * lax.rsqrt(v + eps) * g + b


def _dot(a, b):
    return jnp.dot(a, b, preferred_element_type=F32)


def _enc_front(ea, lng, lnb, w1, b1, cw1, cb1, cw2, cb2):
    ln = _ln_tc(ea, lng[...], lnb[...])
    h = jnp.maximum(_dot(ln, w1[...]) + b1[...], 0.0)
    t = jnp.maximum(_dot(ea, cw1[...]) + cb1[...], 0.0)
    ew = jax.nn.sigmoid(_dot(t, cw2[...]) + cb2[...])
    return h * ew, ew


_ENC_FRONT_SPECS = [(1, 16), (1, 16), (16, 64), (1, 64),
                    (16, 8), (1, 8), (8, 1), (1, 1)]


def _encoder_call(e_pad, tile):
    def body(ea_ref, lng, lnb, w1, b1, cw1, cb1, cw2, cb2,
             wc1, bc1, b11, wc2e, bc2e, b12, wc3, bc3, b13,
             cc_ref, c2_ref, c3_ref):
        ea = ea_ref[...]
        hw, ew = _enc_front(ea, lng, lnb, w1, b1, cw1, cb1, cw2, cb2)
        c1 = _dot(hw, wc1[...]) + bc1[...] * ew + b11[...]
        cc_ref[...] = jnp.concatenate(
            [c1, hw, ew, jnp.zeros((ea.shape[0], 31), F32)], axis=1)
        c2_ref[...] = _dot(hw, wc2e[...]) + bc2e[...] * ew + b12[...]
        c3_ref[...] = _dot(hw, wc3[...]) + bc3[...] * ew + b13[...]

    grid = (e_pad // tile,)
    in_specs = [pl.BlockSpec((tile, 16), lambda i: (i, 0))] + [
        _full(s) for s in _ENC_FRONT_SPECS + [(64, 32), (1, 32), (1, 32),
                                              (64, 64), (1, 64), (1, 64),
                                              (64, 128), (1, 128), (1, 128)]]
    out_specs = [pl.BlockSpec((tile, w), lambda i: (i, 0))
                 for w in (128, 64, 128)]
    out_shape = [jax.ShapeDtypeStruct((e_pad, w), F32) for w in (128, 64, 128)]
    return pl.pallas_call(body, grid=grid, in_specs=in_specs,
                          out_specs=out_specs, out_shape=out_shape)


def _node_pre_call(n_pad, blk):
    def body(x_ref, bg, bb, bm, bv, wab, skw, skb, ab_ref, xp_ref):
        x = (x_ref[...] - bm[...]) * lax.rsqrt(bv[...] + 1e-5) * bg[...] \
            + bb[...]
        ab_ref[...] = _dot(x, wab[...])
        xp_ref[...] = 0.01 * (_dot(x, skw[...]) + skb[...])

    grid = (n_pad // blk,)
    in_specs = [pl.BlockSpec((blk, 128), lambda i: (i, 0))] + [
        _full(s) for s in [(1, 128)] * 4 + [(128, 128), (128, 128),
                                            (1, 128)]]
    out_specs = [pl.BlockSpec((blk, 128), lambda i: (i, 0))] * 2
    out_shape = [jax.ShapeDtypeStruct((n_pad, 128), F32)] * 2
    return pl.pallas_call(body, grid=grid, in_specs=in_specs,
                          out_specs=out_specs, out_shape=out_shape)


def _mid1_call(n_pad, blk):
    def body(u_ref, w2, b2, lng, lnb, wab, ew2, eb2, ab_ref, cnt_ref,
             efs_ref):
        ut = u_ref[0] + u_ref[1]
        u = ut[:, 0:32]
        uhw = ut[:, 32:96]
        sew = ut[:, 96:97]
        cnt = ut[:, 97:98]
        efs_ref[...] = _dot(uhw, ew2[...]) + eb2[...] * sew
        deg = jnp.maximum(cnt, 1.0)
        conv = _dot(u / deg, w2[...]) + b2[...] * jnp.minimum(cnt, 1.0)
        xn = _ln_tc(conv, lng[...], lnb[...])
        xn = jnp.maximum(xn, 0.01 * xn)
        ab_ref[...] = _dot(xn, wab[...])
        cnt_ref[...] = cnt

    grid = (n_pad // blk,)
    in_specs = [pl.BlockSpec((NC, blk, 128), lambda i: (0, i, 0))] + [
        _full(s) for s in [(32, 32), (1, 32), (1, 32), (1, 32), (32, 128),
                           (64, 128), (1, 128)]]
    out_specs = [pl.BlockSpec((blk, 128), lambda i: (i, 0)),
                 pl.BlockSpec((blk, 1), lambda i: (i, 0)),
                 pl.BlockSpec((blk, 128), lambda i: (i, 0))]
    out_shape = [jax.ShapeDtypeStruct((n_pad, 128), F32),
                 jax.ShapeDtypeStruct((n_pad, 1), F32),
                 jax.ShapeDtypeStruct((n_pad, 128), F32)]
    return pl.pallas_call(body, grid=grid, in_specs=in_specs,
                          out_specs=out_specs, out_shape=out_shape)


def _mid2_call(n_pad, blk):
    def body(u_ref, cnt_ref, w2, b2, lng, lnb, wd, ws, a_ref, b_ref):
        u = (u_ref[0] + u_ref[1])[:, 0:64]
        cnt = cnt_ref[...]
        deg = jnp.maximum(cnt, 1.0)
        conv = _dot(u / deg, w2[...]) + b2[...] * jnp.minimum(cnt, 1.0)
        xn = jnp.maximum(_ln_tc(conv, lng[...], lnb[...]), 0.0)
        a_ref[...] = _dot(xn, wd[...])
        b_ref[...] = _dot(xn, ws[...])

    grid = (n_pad // blk,)
    in_specs = [pl.BlockSpec((NC, blk, 128), lambda i: (0, i, 0)),
                pl.BlockSpec((blk, 1), lambda i: (i, 0))] + [
        _full(s) for s in [(64, 64), (1, 64), (1, 64), (1, 64),
                           (64, 128), (64, 128)]]
    out_specs = [pl.BlockSpec((blk, 128), lambda i: (i, 0))] * 2
    out_shape = [jax.ShapeDtypeStruct((n_pad, 128), F32)] * 2
    return pl.pallas_call(body, grid=grid, in_specs=in_specs,
                          out_specs=out_specs, out_shape=out_shape)


def _final_call(n_pad, blk):
    def body(u_ref, cnt_ref, efs_ref, xp_ref, w2, b2, lng, lnb,
             nw1, nb1, nw2, nb2, nw3, nb3, xf_ref, pr_ref):
        u = u_ref[0] + u_ref[1]
        cnt = cnt_ref[...]
        deg = jnp.maximum(cnt, 1.0)
        conv = _dot(u / deg, w2[...]) + b2[...] * jnp.minimum(cnt, 1.0)
        x3 = jnp.maximum(_ln_tc(conv, lng[...], lnb[...]), 0.0)
        xf = jnp.concatenate([xp_ref[...] + x3, efs_ref[...] / deg], axis=1)
        xf_ref[...] = xf
        h = _dot(xf, nw1[...]) + nb1[...]
        h = jnp.where(h > 0, h, jnp.exp(jnp.minimum(h, 0.0)) - 1.0)
        h = _dot(h, nw2[...]) + nb2[...]
        h = jnp.where(h > 0, h, jnp.exp(jnp.minimum(h, 0.0)) - 1.0)
        pr_ref[...] = _dot(h, nw3[...]) + nb3[...]

    grid = (n_pad // blk,)
    in_specs = [pl.BlockSpec((NC, blk, 128), lambda i: (0, i, 0)),
                pl.BlockSpec((blk, 1), lambda i: (i, 0)),
                pl.BlockSpec((blk, 128), lambda i: (i, 0)),
                pl.BlockSpec((blk, 128), lambda i: (i, 0))] + [
        _full(s) for s in [(128, 128), (1, 128), (1, 128), (1, 128),
                           (256, 128), (1, 128), (128, 64), (1, 64),
                           (64, 1), (1, 1)]]
    out_specs = [pl.BlockSpec((blk, 256), lambda i: (i, 0)),
                 pl.BlockSpec((blk, 1), lambda i: (i, 0))]
    out_shape = [jax.ShapeDtypeStruct((n_pad, 256), F32),
                 jax.ShapeDtypeStruct((n_pad, 1), F32)]
    return pl.pallas_call(body, grid=grid, in_specs=in_specs,
                          out_specs=out_specs, out_shape=out_shape)


@jax.jit
def _run(x_in, edge_index, edge_attr, params):
    p = params
    n, _ = x_in.shape
    e = edge_index.shape[1]

    epc = NW * CH * 8
    e_pad = ((e + epc - 1) // epc) * epc
    n_pad = ((n + 8 + 639) // 640) * 640
    nchunks = e_pad // (NW * CH)

    src = jnp.concatenate(
        [edge_index[0], jnp.full((e_pad - e,), n, jnp.int32)])
    dst = jnp.concatenate(
        [edge_index[1], jnp.full((e_pad - e,), n, jnp.int32)])
    ea = jnp.pad(edge_attr, ((0, e_pad - e), (0, 0)))
    x_pad = jnp.pad(x_in, ((0, n_pad - n), (0, 0)))

    def r2(v):
        return v.reshape(1, -1)

    def padw(w, cols):
        return jnp.pad(w, ((0, 0), (0, cols - w.shape[1])))

    wab1 = padw(jnp.concatenate([p['c1_W1'][0:128], p['c1_W1'][128:256]], 1),
                128)
    wab2 = jnp.concatenate([p['c2_W1'][0:32], p['c2_W1'][32:64]], 1)

    enc_w2 = p['enc_W2']
    enc_b2 = r2(p['enc_b2'])

    def hdot(a, b):
        return jnp.dot(a, b, precision=lax.Precision.HIGHEST)

    we1 = hdot(enc_w2, p['c1_W1'][256:384])
    we2 = hdot(enc_w2, p['c2_W1'][64:192])
    we3 = hdot(enc_w2, p['c3_W1'][128:256])
    be1 = hdot(enc_b2, p['c1_W1'][256:384])
    be2 = hdot(enc_b2, p['c2_W1'][64:192])
    be3 = hdot(enc_b2, p['c3_W1'][128:256])

    cc1, c2, c3 = _encoder_call(e_pad, 1024)(
        ea, r2(p['enc_ln_g']), r2(p['enc_ln_b']),
        p['enc_W1'], r2(p['enc_b1']),
        p['cls_W1'], r2(p['cls_b1']), p['cls_W2'], r2(p['cls_b2']),
        we1, be1, r2(p['c1_b1']), we2, be2, r2(p['c2_b1']),
        we3, be3, r2(p['c3_b1']))

    blk = n_pad // 8
    ab1, xp = _node_pre_call(n_pad, blk)(
        x_pad, r2(p['bn0_g']), r2(p['bn0_b']), r2(p['bn0_m']),
        r2(p['bn0_v']), wab1, p['skip_W'], r2(p['skip_b']))

    u1 = _sc_pass(_conv1_compute, 128, nchunks, n_pad)(
        ab1, ab1, cc1, src, dst)

    ab2, cnt, efs = _mid1_call(n_pad, blk)(
        u1, p['c1_W2'], r2(p['c1_b2']), r2(p['ln1_g']), r2(p['ln1_b']),
        wab2, enc_w2, enc_b2)

    u2 = _sc_pass(_conv2_compute, 64, nchunks, n_pad)(
        ab2, ab2, c2, src, dst)

    a3, b3 = _mid2_call(n_pad, blk)(
        u2, cnt, p['c2_W2'], r2(p['c2_b2']), r2(p['ln2_g']), r2(p['ln2_b']),
        p['c3_W1'][0:64], p['c3_W1'][64:128])

    u3 = _sc_pass(_conv3_compute, 128, nchunks, n_pad)(
        a3, b3, c3, src, dst)

    xf, probs = _final_call(n_pad, blk)(
        u3, cnt, efs, xp, p['c3_W2'], r2(p['c3_b2']),
        r2(p['ln3_g']), r2(p['ln3_b']),
        p['np_W1'], r2(p['np_b1']), p['np_W2'], r2(p['np_b2']),
        p['np_W3'], r2(p['np_b3']))

    return xf[:n], probs[:n]


def kernel(x_in, edge_index, edge_attr, params):
    xf, probs = _run(x_in, edge_index, edge_attr, params)
    return (xf, probs, None, None, None)
